# baseline (speedup 1.0000x reference)
import functools

import jax
import jax.numpy as jnp
from jax import lax
from jax.experimental import pallas as pl
from jax.experimental.pallas import tpu as pltpu
from jax.experimental.pallas import tpu_sc as plsc

NUM_FEATURES = 768
HIDDEN = 512
MAX_ACTIVE = 32
BATCH = 4096

NUM_TILES = 32
ROWS_PER_TILE = BATCH // NUM_TILES
CHUNK_ROWS = 64
NUM_CHUNKS = ROWS_PER_TILE // CHUNK_ROWS
WORDS = 512

BB = 1024


def _make_sc_counts_body(batch, row_offset):
    rows_per_tile = batch // NUM_TILES
    num_chunks = max(1, rows_per_tile // CHUNK_ROWS)
    chunk_rows = rows_per_tile // num_chunks

    slab = max(rows_per_tile, 128)
    tiles_per_slab = slab // rows_per_tile

    def body(wf_hbm, bf_hbm, counts_hbm,
             featw_v, featb_v, counts_a, counts_b, sem_a, sem_b):
        wid = lax.axis_index("s") * 2 + lax.axis_index("c")
        slab_base = row_offset + (wid // tiles_per_slab) * slab
        sub = (wid % tiles_per_slab) * rows_per_tile
        pltpu.sync_copy(wf_hbm.at[:, pl.ds(slab_base, slab)], featw_v)
        pltpu.sync_copy(bf_hbm.at[:, pl.ds(slab_base, slab)], featb_v)

        lane = lax.iota(jnp.int32, 16)
        izeros = jnp.zeros((16,), jnp.int32)
        ones = jnp.ones((16,), jnp.int32)
        eights = jnp.full((16,), 8, jnp.int32)

        bufs = (counts_a, counts_b)
        sems = (sem_a, sem_b)
        copies = [None] * num_chunks
        for chunk in range(num_chunks):
            counts_v = bufs[chunk % 2]
            if chunk >= 2:
                copies[chunk - 2].wait()

            def zero_row(r, carry):
                for c in range(WORDS // 16):
                    counts_v[r, pl.ds(c * 16, 16)] = izeros
                return carry
            lax.fori_loop(0, chunk_rows, zero_row, 0)

            def scatter_group(g, carry):
                crow = g * 16 + lane
                foff = sub + chunk * chunk_rows + g * 16
                for a in range(MAX_ACTIVE):
                    fw = featw_v[a, pl.ds(foff, 16)]
                    val_w = ones << ((fw >> 9) * eights)
                    plsc.addupdate_scatter(counts_v, [crow, fw & (WORDS - 1)], val_w)
                    gb = featb_v[a, pl.ds(foff, 16)] + 1024
                    val_b = ones << ((gb >> 9) * eights)
                    plsc.addupdate_scatter(counts_v, [crow, gb & (WORDS - 1)], val_b)
                return carry
            lax.fori_loop(0, chunk_rows // 16, scatter_group, 0)

            copies[chunk] = pltpu.make_async_copy(
                counts_v,
                counts_hbm.at[pl.ds(wid * rows_per_tile + chunk * chunk_rows,
                                    chunk_rows), :],
                sems[chunk % 2])
            copies[chunk].start()
        for chunk in range(max(0, num_chunks - 2), num_chunks):
            copies[chunk].wait()

    return body, rows_per_tile, chunk_rows


def _sc_counts(white_features, black_features, batch, row_offset):
    body, rows_per_tile, chunk_rows = _make_sc_counts_body(batch, row_offset)
    mesh = plsc.VectorSubcoreMesh(core_axis_name="c", subcore_axis_name="s")
    k = pl.kernel(
        body,
        out_type=jax.ShapeDtypeStruct((batch, WORDS), jnp.int32),
        mesh=mesh,
        compiler_params=pltpu.CompilerParams(needs_layout_passes=False),
        scratch_types=[
            pltpu.VMEM((MAX_ACTIVE, max(rows_per_tile, 128)), jnp.int32),
            pltpu.VMEM((MAX_ACTIVE, max(rows_per_tile, 128)), jnp.int32),
            pltpu.VMEM((chunk_rows, WORDS), jnp.int32),
            pltpu.VMEM((chunk_rows, WORDS), jnp.int32),
            pltpu.SemaphoreType.DMA,
            pltpu.SemaphoreType.DMA,
        ],
    )
    return k(white_features, black_features)


def _tc_dense_body(counts_ref, stm_ref, table_ref, bias_ref, ow_ref, ob_ref, out_ref):
    w = counts_ref[...]
    t_lo = table_ref[:WORDS, :].astype(jnp.bfloat16)
    t_hi = table_ref[WORDS:, :].astype(jnp.bfloat16)
    bias = bias_ref[0, :][None, :]

    def acc_of(p_lo, p_hi):
        f_lo = (p_lo & 255).astype(jnp.bfloat16)
        f_hi = (p_hi & 255).astype(jnp.bfloat16)
        return (jnp.dot(f_lo, t_lo, preferred_element_type=jnp.float32)
                + jnp.dot(f_hi[:, :NUM_FEATURES - WORDS], t_hi,
                          preferred_element_type=jnp.float32) + bias)

    acc_w = acc_of(w, w >> 8)
    acc_b = acc_of(w >> 16, w >> 24)

    act_w = jnp.square(jnp.clip(acc_w, 0.0, 1.0))
    act_b = jnp.square(jnp.clip(acc_b, 0.0, 1.0))

    w_us = ow_ref[0, :HIDDEN][None, :]
    w_them = ow_ref[0, HIDDEN:][None, :]
    u = (w_us + w_them) * 0.5
    v = (w_us - w_them) * 0.5
    sp = act_w + act_b
    dv = (act_w - act_b) * v
    r_f = jnp.sum(dv, axis=1)
    r_e = jnp.sum(sp * u + dv, axis=1)
    s = stm_ref[...].astype(jnp.float32)
    out_ref[...] = r_e - 2.0 * s * r_f + ob_ref[0, 0]


def _tc_dense(counts, stm, ft_weight, ft_bias, out_weight, out_bias):
    batch = counts.shape[0]
    grid = (batch // BB,)
    return pl.pallas_call(
        _tc_dense_body,
        grid=grid,
        in_specs=[
            pl.BlockSpec((BB, WORDS), lambda i: (i, 0)),
            pl.BlockSpec((BB,), lambda i: (i,)),
            pl.BlockSpec((NUM_FEATURES, HIDDEN), lambda i: (0, 0)),
            pl.BlockSpec((1, HIDDEN), lambda i: (0, 0)),
            pl.BlockSpec((1, 2 * HIDDEN), lambda i: (0, 0)),
            pl.BlockSpec((1, 1), lambda i: (0, 0)),
        ],
        out_specs=pl.BlockSpec((BB,), lambda i: (i,)),
        out_shape=jax.ShapeDtypeStruct((batch,), jnp.float32),
    )(
        counts,
        stm,
        ft_weight,
        ft_bias[None, :],
        out_weight[None, :],
        out_bias[None, :],
    )


def kernel(white_features, black_features, stm, ft_weight, ft_bias, out_weight, out_bias):
    wft = white_features.T
    bft = black_features.T
    half = BATCH // 2
    c1 = _sc_counts(wft, bft, half, 0)
    c2 = _sc_counts(wft, bft, half, half)
    o1 = _tc_dense(c1, stm[:half], ft_weight, ft_bias, out_weight, out_bias)
    o2 = _tc_dense(c2, stm[half:], ft_weight, ft_bias, out_weight, out_bias)
    return jnp.concatenate([o1, o2])


# speedup vs baseline: 1.0380x; 1.0380x over previous
import functools

import jax
import jax.numpy as jnp
from jax import lax
from jax.experimental import pallas as pl
from jax.experimental.pallas import tpu as pltpu
from jax.experimental.pallas import tpu_sc as plsc

NUM_FEATURES = 768
HIDDEN = 512
MAX_ACTIVE = 32
BATCH = 4096

NUM_TILES = 32
ROWS_PER_TILE = BATCH // NUM_TILES
CHUNK_ROWS = 64
NUM_CHUNKS = ROWS_PER_TILE // CHUNK_ROWS
WORDS = 512

BB = 512


def _make_sc_counts_body(batch, row_offset):
    rows_per_tile = batch // NUM_TILES
    num_chunks = max(1, rows_per_tile // CHUNK_ROWS)
    chunk_rows = rows_per_tile // num_chunks

    slab = max(rows_per_tile, 128)
    tiles_per_slab = slab // rows_per_tile

    def body(wf_hbm, bf_hbm, counts_hbm,
             featw_v, featb_v, counts_a, counts_b, sem_a, sem_b, sem_f):
        wid = lax.axis_index("s") * 2 + lax.axis_index("c")
        slab_base = row_offset + (wid // tiles_per_slab) * slab
        sub = (wid % tiles_per_slab) * rows_per_tile
        fcw = pltpu.make_async_copy(
            wf_hbm.at[:, pl.ds(slab_base, slab)], featw_v, sem_f)
        fcb = pltpu.make_async_copy(
            bf_hbm.at[:, pl.ds(slab_base, slab)], featb_v, sem_f)
        fcw.start()
        fcb.start()

        lane = lax.iota(jnp.int32, 16)
        izeros = jnp.zeros((16,), jnp.int32)
        ones = jnp.ones((16,), jnp.int32)
        eights = jnp.full((16,), 8, jnp.int32)

        bufs = (counts_a, counts_b)
        sems = (sem_a, sem_b)
        copies = [None] * num_chunks
        for chunk in range(num_chunks):
            counts_v = bufs[chunk % 2]
            if chunk >= 2:
                copies[chunk - 2].wait()

            def zero_row(r, carry):
                for c in range(WORDS // 16):
                    counts_v[r, pl.ds(c * 16, 16)] = izeros
                return carry
            lax.fori_loop(0, chunk_rows, zero_row, 0)

            if chunk == 0:
                fcw.wait()
                fcb.wait()

            def scatter_group(g, carry):
                crow = g * 16 + lane
                foff = sub + chunk * chunk_rows + g * 16
                for a in range(MAX_ACTIVE):
                    fw = featw_v[a, pl.ds(foff, 16)]
                    val_w = ones << ((fw >> 9) * eights)
                    plsc.addupdate_scatter(counts_v, [crow, fw & (WORDS - 1)], val_w)
                    gb = featb_v[a, pl.ds(foff, 16)] + 1024
                    val_b = ones << ((gb >> 9) * eights)
                    plsc.addupdate_scatter(counts_v, [crow, gb & (WORDS - 1)], val_b)
                return carry
            lax.fori_loop(0, chunk_rows // 16, scatter_group, 0)

            copies[chunk] = pltpu.make_async_copy(
                counts_v,
                counts_hbm.at[pl.ds(wid * rows_per_tile + chunk * chunk_rows,
                                    chunk_rows), :],
                sems[chunk % 2])
            copies[chunk].start()
        for chunk in range(max(0, num_chunks - 2), num_chunks):
            copies[chunk].wait()

    return body, rows_per_tile, chunk_rows


def _sc_counts(white_features, black_features, batch, row_offset):
    body, rows_per_tile, chunk_rows = _make_sc_counts_body(batch, row_offset)
    mesh = plsc.VectorSubcoreMesh(core_axis_name="c", subcore_axis_name="s")
    k = pl.kernel(
        body,
        out_type=jax.ShapeDtypeStruct((batch, WORDS), jnp.int32),
        mesh=mesh,
        compiler_params=pltpu.CompilerParams(needs_layout_passes=False),
        scratch_types=[
            pltpu.VMEM((MAX_ACTIVE, max(rows_per_tile, 128)), jnp.int32),
            pltpu.VMEM((MAX_ACTIVE, max(rows_per_tile, 128)), jnp.int32),
            pltpu.VMEM((chunk_rows, WORDS), jnp.int32),
            pltpu.VMEM((chunk_rows, WORDS), jnp.int32),
            pltpu.SemaphoreType.DMA,
            pltpu.SemaphoreType.DMA,
            pltpu.SemaphoreType.DMA,
        ],
    )
    return k(white_features, black_features)


def _tc_dense_body(counts_ref, stm_ref, table_ref, bias_ref, ow_ref, ob_ref, out_ref):
    w = counts_ref[...]
    t_lo = table_ref[:WORDS, :].astype(jnp.bfloat16)
    t_hi = table_ref[WORDS:, :].astype(jnp.bfloat16)
    bias = bias_ref[0, :][None, :]

    def acc_of(p_lo, p_hi):
        f_lo = (p_lo & 255).astype(jnp.bfloat16)
        f_hi = (p_hi & 255).astype(jnp.bfloat16)
        return (jnp.dot(f_lo, t_lo, preferred_element_type=jnp.float32)
                + jnp.dot(f_hi[:, :NUM_FEATURES - WORDS], t_hi,
                          preferred_element_type=jnp.float32) + bias)

    acc_w = acc_of(w, w >> 8)
    acc_b = acc_of(w >> 16, w >> 24)

    act_w = jnp.square(jnp.clip(acc_w, 0.0, 1.0))
    act_b = jnp.square(jnp.clip(acc_b, 0.0, 1.0))

    w_us = ow_ref[0, :HIDDEN][None, :]
    w_them = ow_ref[0, HIDDEN:][None, :]
    u = (w_us + w_them) * 0.5
    v = (w_us - w_them) * 0.5
    sp = act_w + act_b
    dv = (act_w - act_b) * v
    r_f = jnp.sum(dv, axis=1)
    r_e = jnp.sum(sp * u + dv, axis=1)
    s = stm_ref[...].astype(jnp.float32)
    out_ref[...] = r_e - 2.0 * s * r_f + ob_ref[0, 0]


def _tc_dense(counts, stm, ft_weight, ft_bias, out_weight, out_bias):
    batch = counts.shape[0]
    grid = (batch // BB,)
    return pl.pallas_call(
        _tc_dense_body,
        grid=grid,
        in_specs=[
            pl.BlockSpec((BB, WORDS), lambda i: (i, 0)),
            pl.BlockSpec((BB,), lambda i: (i,)),
            pl.BlockSpec((NUM_FEATURES, HIDDEN), lambda i: (0, 0)),
            pl.BlockSpec((1, HIDDEN), lambda i: (0, 0)),
            pl.BlockSpec((1, 2 * HIDDEN), lambda i: (0, 0)),
            pl.BlockSpec((1, 1), lambda i: (0, 0)),
        ],
        out_specs=pl.BlockSpec((BB,), lambda i: (i,)),
        out_shape=jax.ShapeDtypeStruct((batch,), jnp.float32),
    )(
        counts,
        stm,
        ft_weight,
        ft_bias[None, :],
        out_weight[None, :],
        out_bias[None, :],
    )


def kernel(white_features, black_features, stm, ft_weight, ft_bias, out_weight, out_bias):
    counts = _sc_counts(white_features.T, black_features.T, BATCH, 0)
    return _tc_dense(counts, stm, ft_weight, ft_bias, out_weight, out_bias)


# speedup vs baseline: 1.0381x; 1.0001x over previous
import jax
import jax.numpy as jnp
from jax import lax
from jax.experimental import pallas as pl
from jax.experimental.pallas import tpu as pltpu
from jax.experimental.pallas import tpu_sc as plsc

NUM_FEATURES = 768
HIDDEN = 512
MAX_ACTIVE = 32
BATCH = 4096

NUM_TILES = 32
CHUNK_ROWS = 64
WORDS = 512

BB = 512


def _make_sc_counts_body(batch, row_offset):
    rows_per_tile = batch // NUM_TILES
    num_chunks = max(1, rows_per_tile // CHUNK_ROWS)
    chunk_rows = rows_per_tile // num_chunks

    slab = max(rows_per_tile, 128)
    tiles_per_slab = slab // rows_per_tile

    def body(wf_hbm, bf_hbm, counts_hbm,
             featw_v, featb_v, counts_a, counts_b, sem_a, sem_b, sem_f):
        wid = lax.axis_index("s") * 2 + lax.axis_index("c")
        slab_base = row_offset + (wid // tiles_per_slab) * slab
        sub = (wid % tiles_per_slab) * rows_per_tile
        fcw = pltpu.make_async_copy(
            wf_hbm.at[:, pl.ds(slab_base, slab)], featw_v, sem_f)
        fcb = pltpu.make_async_copy(
            bf_hbm.at[:, pl.ds(slab_base, slab)], featb_v, sem_f)
        fcw.start()
        fcb.start()

        lane = lax.iota(jnp.int32, 16)
        izeros = jnp.zeros((16,), jnp.int32)
        ones = jnp.ones((16,), jnp.int32)
        eights = jnp.full((16,), 8, jnp.int32)

        bufs = (counts_a, counts_b)
        sems = (sem_a, sem_b)
        copies = [None] * num_chunks
        for chunk in range(num_chunks):
            counts_v = bufs[chunk % 2]
            if chunk >= 2:
                copies[chunk - 2].wait()

            def zero_row(r, carry):
                for c in range(WORDS // 16):
                    counts_v[r, pl.ds(c * 16, 16)] = izeros
                return carry
            lax.fori_loop(0, chunk_rows, zero_row, 0)

            if chunk == 0:
                fcw.wait()
                fcb.wait()

            def scatter_group(g, carry):
                crow = g * 16 + lane
                foff = sub + chunk * chunk_rows + g * 16
                for a in range(MAX_ACTIVE):
                    fw = featw_v[a, pl.ds(foff, 16)]
                    val_w = ones << ((fw >> 9) * eights)
                    plsc.addupdate_scatter(counts_v, [crow, fw & (WORDS - 1)], val_w)
                    gb = featb_v[a, pl.ds(foff, 16)] + 1024
                    val_b = ones << ((gb >> 9) * eights)
                    plsc.addupdate_scatter(counts_v, [crow, gb & (WORDS - 1)], val_b)
                return carry
            lax.fori_loop(0, chunk_rows // 16, scatter_group, 0)

            copies[chunk] = pltpu.make_async_copy(
                counts_v,
                counts_hbm.at[pl.ds(wid * rows_per_tile + chunk * chunk_rows,
                                    chunk_rows), :],
                sems[chunk % 2])
            copies[chunk].start()
        for chunk in range(max(0, num_chunks - 2), num_chunks):
            copies[chunk].wait()

    return body, rows_per_tile, chunk_rows


def _sc_counts(white_features, black_features, batch, row_offset):
    body, rows_per_tile, chunk_rows = _make_sc_counts_body(batch, row_offset)
    mesh = plsc.VectorSubcoreMesh(core_axis_name="c", subcore_axis_name="s")
    k = pl.kernel(
        body,
        out_type=jax.ShapeDtypeStruct((batch, WORDS), jnp.int32),
        mesh=mesh,
        compiler_params=pltpu.CompilerParams(needs_layout_passes=False),
        scratch_types=[
            pltpu.VMEM((MAX_ACTIVE, max(rows_per_tile, 128)), jnp.int32),
            pltpu.VMEM((MAX_ACTIVE, max(rows_per_tile, 128)), jnp.int32),
            pltpu.VMEM((chunk_rows, WORDS), jnp.int32),
            pltpu.VMEM((chunk_rows, WORDS), jnp.int32),
            pltpu.SemaphoreType.DMA,
            pltpu.SemaphoreType.DMA,
            pltpu.SemaphoreType.DMA,
        ],
    )
    return k(white_features, black_features)


def _tc_dense_body(counts_ref, stm_ref, table_ref, bias_ref, ow_ref, ob_ref, out_ref):
    w = counts_ref[...]
    t_lo = table_ref[:WORDS, :].astype(jnp.bfloat16)
    t_hi = table_ref[WORDS:, :].astype(jnp.bfloat16)
    bias = bias_ref[0, :][None, :]

    def acc_of(p_lo, p_hi):
        f_lo = (p_lo & 255).astype(jnp.bfloat16)
        f_hi = (p_hi & 255).astype(jnp.bfloat16)
        return (jnp.dot(f_lo, t_lo, preferred_element_type=jnp.float32)
                + jnp.dot(f_hi[:, :NUM_FEATURES - WORDS], t_hi,
                          preferred_element_type=jnp.float32) + bias)

    acc_w = acc_of(w, w >> 8)
    acc_b = acc_of(w >> 16, w >> 24)

    act_w = jnp.square(jnp.clip(acc_w, 0.0, 1.0))
    act_b = jnp.square(jnp.clip(acc_b, 0.0, 1.0))

    w_us = ow_ref[0, :HIDDEN][None, :]
    w_them = ow_ref[0, HIDDEN:][None, :]
    u = (w_us + w_them) * 0.5
    v = (w_us - w_them) * 0.5
    sp = act_w + act_b
    dv = (act_w - act_b) * v
    r_f = jnp.sum(dv, axis=1)
    r_e = jnp.sum(sp * u + dv, axis=1)
    s = stm_ref[...].astype(jnp.float32)
    out_ref[...] = r_e - 2.0 * s * r_f + ob_ref[0, 0]


def _tc_dense(counts, stm, ft_weight, ft_bias, out_weight, out_bias):
    batch = counts.shape[0]
    grid = (batch // BB,)
    return pl.pallas_call(
        _tc_dense_body,
        grid=grid,
        in_specs=[
            pl.BlockSpec((BB, WORDS), lambda i: (i, 0)),
            pl.BlockSpec((BB,), lambda i: (i,)),
            pl.BlockSpec((NUM_FEATURES, HIDDEN), lambda i: (0, 0)),
            pl.BlockSpec((1, HIDDEN), lambda i: (0, 0)),
            pl.BlockSpec((1, 2 * HIDDEN), lambda i: (0, 0)),
            pl.BlockSpec((1, 1), lambda i: (0, 0)),
        ],
        out_specs=pl.BlockSpec((BB,), lambda i: (i,)),
        out_shape=jax.ShapeDtypeStruct((batch,), jnp.float32),
    )(
        counts,
        stm,
        ft_weight,
        ft_bias[None, :],
        out_weight[None, :],
        out_bias[None, :],
    )


def kernel(white_features, black_features, stm, ft_weight, ft_bias, out_weight, out_bias):
    counts = _sc_counts(white_features.T, black_features.T, BATCH, 0)
    return _tc_dense(counts, stm, ft_weight, ft_bias, out_weight, out_bias)
